# baseline (speedup 1.0000x reference)
import functools

import jax
import jax.numpy as jnp
from jax import lax
from jax.experimental import pallas as pl
from jax.experimental.pallas import tpu as pltpu
from jax.experimental.pallas import tpu_sc as plsc

NODE_SIZE = 100000
PATH_SIZE = 64
EMBED_DIM = 128
NUM_ETYPES = 4
NEI = 16
BATCH = 1024

NC = 2
NS = 16
NW = NC * NS
BPW = BATCH // NW
ROWS = NUM_ETYPES * NEI
KB = 8


def _sc_body(nbr_hbm, sidx_hbm, eidx_hbm, pidx_hbm, estart_hbm, eend_hbm,
             epath_hbm, agg_hbm, eemb_hbm, praw_hbm,
             idx_v, ridx_v, eidx_v, pidx_v, nbr_v, sel_vs, rows_vs, out_v,
             eemb_v, pemb_v, sem_i, sems, sem_m, sem_p):
    wid = lax.axis_index("s") * NC + lax.axis_index("c")
    base = wid * BPW

    pltpu.sync_copy(sidx_hbm.at[pl.ds(base, BPW)], idx_v)
    pltpu.sync_copy(eidx_hbm.at[pl.ds(base, BPW)], eidx_v)
    pltpu.sync_copy(pidx_hbm.at[pl.ds(base, BPW)], pidx_v)
    pltpu.async_copy(eend_hbm.at[eidx_v], eemb_v, sem_m)
    pltpu.async_copy(epath_hbm.at[pidx_v], pemb_v, sem_p)

    for c in range(BPW // 16):
        sl = pl.ds(c * 16, 16)
        ridx_v[sl] = lax.shift_right_logical(idx_v[sl], 1)
    pltpu.async_copy(nbr_hbm.at[ridx_v], nbr_v, sem_i).wait()

    def select(j, b):
        sj = plsc.load_gather(idx_v, [jnp.full((16,), j, jnp.int32)])
        par = (sj & 1) == 1
        for c in range(ROWS // 16):
            lo = nbr_v[j, pl.ds(c * 16, 16)]
            hi = nbr_v[j, pl.ds(ROWS + c * 16, 16)]
            sel_vs[b][pl.ds(c * 16, 16)] = jnp.where(par, hi, lo)

    def accum(j, b):
        for e in range(NUM_ETYPES):
            for c in range(EMBED_DIM // 16):
                sl = pl.ds(c * 16, 16)
                vals = [rows_vs[b][e * NEI + r, sl] for r in range(NEI)]
                while len(vals) > 1:
                    vals = [vals[i] + vals[i + 1]
                            for i in range(0, len(vals), 2)]
                out_v[e, j, sl] = vals[0] * (1.0 / NEI)

    def body(h, carry):
        j0 = h * KB
        for b in range(KB):
            select(j0 + b, b)
            pltpu.async_copy(estart_hbm.at[sel_vs[b]], rows_vs[b], sems[b])
        for b in range(KB):
            pltpu.make_async_copy(estart_hbm.at[sel_vs[b]], rows_vs[b],
                                  sems[b]).wait()
            accum(j0 + b, b)
        return carry

    lax.fori_loop(0, BPW // KB, body, 0)

    for e in range(NUM_ETYPES):
        pltpu.sync_copy(out_v.at[e], agg_hbm.at[e, pl.ds(base, BPW)])
    pltpu.make_async_copy(eend_hbm.at[eidx_v], eemb_v, sem_m).wait()
    pltpu.sync_copy(eemb_v, eemb_hbm.at[pl.ds(base, BPW)])
    pltpu.make_async_copy(epath_hbm.at[pidx_v], pemb_v, sem_p).wait()
    pltpu.sync_copy(pemb_v, praw_hbm.at[pl.ds(base, BPW)])


def _sc_entry(nbr_hbm, sidx_hbm, eidx_hbm, pidx_hbm, estart_hbm, eend_hbm,
              epath_hbm, agg_hbm, eemb_hbm, praw_hbm,
              idx_v, ridx_v, eidx_v, pidx_v, nbr_v,
              s0, s1, s2, s3, s4, s5, s6, s7,
              r0, r1, r2, r3, r4, r5, r6, r7,
              out_v, eemb_v, pemb_v, sem_i,
              d0, d1, d2, d3, d4, d5, d6, d7, sem_m, sem_p):
    _sc_body(nbr_hbm, sidx_hbm, eidx_hbm, pidx_hbm, estart_hbm, eend_hbm,
             epath_hbm, agg_hbm, eemb_hbm, praw_hbm,
             idx_v, ridx_v, eidx_v, pidx_v, nbr_v,
             (s0, s1, s2, s3, s4, s5, s6, s7),
             (r0, r1, r2, r3, r4, r5, r6, r7),
             out_v, eemb_v, pemb_v, sem_i,
             (d0, d1, d2, d3, d4, d5, d6, d7), sem_m, sem_p)


_sc_gather = functools.partial(
    pl.kernel,
    out_type=(
        jax.ShapeDtypeStruct((NUM_ETYPES, BATCH, EMBED_DIM), jnp.float32),
        jax.ShapeDtypeStruct((BATCH, EMBED_DIM), jnp.float32),
        jax.ShapeDtypeStruct((BATCH, EMBED_DIM), jnp.float32),
    ),
    mesh=plsc.VectorSubcoreMesh(
        core_axis_name="c", subcore_axis_name="s", num_cores=NC,
        num_subcores=NS),
    compiler_params=pltpu.CompilerParams(needs_layout_passes=False),
    scratch_types=(
        [pltpu.VMEM((BPW,), jnp.int32)] * 4
        + [pltpu.VMEM((BPW, 2 * ROWS), jnp.int32)]
        + [pltpu.VMEM((ROWS,), jnp.int32)] * KB
        + [pltpu.VMEM((ROWS, EMBED_DIM), jnp.float32)] * KB
        + [pltpu.VMEM((NUM_ETYPES, BPW, EMBED_DIM), jnp.float32)]
        + [pltpu.VMEM((BPW, EMBED_DIM), jnp.float32)] * 2
        + [pltpu.SemaphoreType.DMA] * (KB + 3)
    ),
)(_sc_entry)


def _tc_body(agg_ref, eemb_ref, praw_ref, W1_ref, b1_ref, W2_ref, b2_ref,
             out_ref):
    f32 = jnp.float32
    hi = lax.Precision.HIGHEST
    W1 = W1_ref[...]
    b1 = b1_ref[...]
    acc = jnp.broadcast_to(b2_ref[...], (BATCH, EMBED_DIM))
    for e in range(NUM_ETYPES):
        h = lax.dot_general(agg_ref[e], W1, (((1,), (0,)), ((), ())),
                            precision=hi, preferred_element_type=f32) + b1
        W2e = W2_ref[pl.ds(e * EMBED_DIM, EMBED_DIM), :]
        acc = acc + lax.dot_general(h, W2e, (((1,), (0,)), ((), ())),
                                    precision=hi, preferred_element_type=f32)
    p = jax.nn.sigmoid(praw_ref[...])
    m = acc * eemb_ref[...] * p
    out_ref[...] = jax.nn.sigmoid(jnp.sum(m, axis=1, keepdims=True))


_tc_dense = pl.pallas_call(
    _tc_body,
    out_shape=jax.ShapeDtypeStruct((BATCH, 1), jnp.float32),
)


def kernel(neighbors, start_node, end_node, path, embeds_start, embeds_end,
           embeds_path, W1, b1, W2, b2):
    nbr_flat = neighbors.reshape(NODE_SIZE // 2, 2 * ROWS)
    agg, eemb, praw = _sc_gather(
        nbr_flat, start_node.astype(jnp.int32), end_node.astype(jnp.int32),
        path.astype(jnp.int32), embeds_start, embeds_end, embeds_path)
    out = _tc_dense(agg, eemb, praw, W1, b1.reshape(1, EMBED_DIM), W2,
                    b2.reshape(1, EMBED_DIM))
    return out.reshape(BATCH)


# speedup vs baseline: 2.7234x; 2.7234x over previous
import functools

import jax
import jax.numpy as jnp
from jax import lax
from jax.experimental import pallas as pl
from jax.experimental.pallas import tpu as pltpu
from jax.experimental.pallas import tpu_sc as plsc

NODE_SIZE = 100000
PATH_SIZE = 64
EMBED_DIM = 128
NUM_ETYPES = 4
NEI = 16
BATCH = 1024

NC = 2
NS = 16
NW = NC * NS
BPW = BATCH // NW
ROWS = NUM_ETYPES * NEI
KB = 4


def _sc_body(nbrT_hbm, sidx_hbm, eidx_hbm, pidx_hbm, estart_hbm, eend_hbm,
             epath_hbm, agg_hbm, eemb_hbm, praw_hbm,
             idx_v, eidx_v, pidx_v, nbr_vs, sel_vs, rows_vs, out_v,
             eemb_v, pemb_v, sem_i, sems, sem_m, sem_p):
    wid = lax.axis_index("s") * NC + lax.axis_index("c")
    base = wid * BPW

    pltpu.sync_copy(sidx_hbm.at[pl.ds(base, BPW)], idx_v)
    pltpu.sync_copy(eidx_hbm.at[pl.ds(base, BPW)], eidx_v)
    pltpu.sync_copy(pidx_hbm.at[pl.ds(base, BPW)], pidx_v)
    pltpu.async_copy(eend_hbm.at[eidx_v], eemb_v, sem_m)
    pltpu.async_copy(epath_hbm.at[pidx_v], pemb_v, sem_p)

    def accum(j, g):
        for e in range(NUM_ETYPES):
            for c in range(EMBED_DIM // 16):
                sl = pl.ds(c * 16, 16)
                vals = [rows_vs[g][e * NEI + r, sl] for r in range(NEI)]
                while len(vals) > 1:
                    vals = [vals[i] + vals[i + 1]
                            for i in range(0, len(vals), 2)]
                out_v[e, j, sl] = vals[0] * (1.0 / NEI)

    iota16 = lax.broadcasted_iota(jnp.int32, (16,), 0)

    def body(h, carry):
        j0 = h * KB
        sjvs = []
        for g in range(KB):
            sjv = plsc.load_gather(
                idx_v, [jnp.full((16,), j0 + g, jnp.int32)])
            sjvs.append(sjv)
            t128 = pl.multiple_of(
                lax.shift_left(lax.shift_right_logical(sjv[0], 7), 7), 128)
            pltpu.async_copy(nbrT_hbm.at[:, :, pl.ds(t128, 128)],
                             nbr_vs[g], sem_i)
        for g in range(KB):
            pltpu.make_async_copy(
                nbrT_hbm.at[:, :, pl.ds(0, 128)], nbr_vs[g], sem_i).wait()
            q = sjvs[g] & 127
            for e in range(NUM_ETYPES):
                ids16 = plsc.load_gather(
                    nbr_vs[g], [jnp.full((16,), e, jnp.int32), iota16, q])
                sel_vs[g][pl.ds(e * NEI, 16)] = ids16
            pltpu.async_copy(estart_hbm.at[sel_vs[g]], rows_vs[g], sems[g])
        for g in range(KB):
            pltpu.make_async_copy(estart_hbm.at[sel_vs[g]], rows_vs[g],
                                  sems[g]).wait()
            accum(j0 + g, g)
        return carry

    lax.fori_loop(0, BPW // KB, body, 0)

    for e in range(NUM_ETYPES):
        pltpu.sync_copy(out_v.at[e], agg_hbm.at[e, pl.ds(base, BPW)])
    pltpu.make_async_copy(eend_hbm.at[eidx_v], eemb_v, sem_m).wait()
    pltpu.sync_copy(eemb_v, eemb_hbm.at[pl.ds(base, BPW)])
    pltpu.make_async_copy(epath_hbm.at[pidx_v], pemb_v, sem_p).wait()
    pltpu.sync_copy(pemb_v, praw_hbm.at[pl.ds(base, BPW)])


def _sc_entry(nbrT_hbm, sidx_hbm, eidx_hbm, pidx_hbm, estart_hbm, eend_hbm,
              epath_hbm, agg_hbm, eemb_hbm, praw_hbm,
              idx_v, eidx_v, pidx_v,
              n0, n1, n2, n3,
              c0, c1, c2, c3,
              r0, r1, r2, r3,
              out_v, eemb_v, pemb_v, sem_i,
              d0, d1, d2, d3, sem_m, sem_p):
    _sc_body(nbrT_hbm, sidx_hbm, eidx_hbm, pidx_hbm, estart_hbm, eend_hbm,
             epath_hbm, agg_hbm, eemb_hbm, praw_hbm,
             idx_v, eidx_v, pidx_v, (n0, n1, n2, n3), (c0, c1, c2, c3),
             (r0, r1, r2, r3),
             out_v, eemb_v, pemb_v, sem_i, (d0, d1, d2, d3), sem_m, sem_p)


_sc_gather = functools.partial(
    pl.kernel,
    out_type=(
        jax.ShapeDtypeStruct((NUM_ETYPES, BATCH, EMBED_DIM), jnp.float32),
        jax.ShapeDtypeStruct((BATCH, EMBED_DIM), jnp.float32),
        jax.ShapeDtypeStruct((BATCH, EMBED_DIM), jnp.float32),
    ),
    mesh=plsc.VectorSubcoreMesh(
        core_axis_name="c", subcore_axis_name="s", num_cores=NC,
        num_subcores=NS),
    compiler_params=pltpu.CompilerParams(needs_layout_passes=False),
    scratch_types=(
        [pltpu.VMEM((BPW,), jnp.int32)] * 3
        + [pltpu.VMEM((NUM_ETYPES, NEI, 128), jnp.int32)] * KB
        + [pltpu.VMEM((ROWS,), jnp.int32)] * KB
        + [pltpu.VMEM((ROWS, EMBED_DIM), jnp.float32)] * KB
        + [pltpu.VMEM((NUM_ETYPES, BPW, EMBED_DIM), jnp.float32)]
        + [pltpu.VMEM((BPW, EMBED_DIM), jnp.float32)] * 2
        + [pltpu.SemaphoreType.DMA] * (KB + 3)
    ),
)(_sc_entry)


def _tc_body(agg_ref, eemb_ref, praw_ref, W1_ref, b1_ref, W2_ref, b2_ref,
             out_ref):
    f32 = jnp.float32
    hi = lax.Precision.HIGHEST
    W1 = W1_ref[...]
    b1 = b1_ref[...]
    acc = jnp.broadcast_to(b2_ref[...], (BATCH, EMBED_DIM))
    for e in range(NUM_ETYPES):
        h = lax.dot_general(agg_ref[e], W1, (((1,), (0,)), ((), ())),
                            precision=hi, preferred_element_type=f32) + b1
        W2e = W2_ref[pl.ds(e * EMBED_DIM, EMBED_DIM), :]
        acc = acc + lax.dot_general(h, W2e, (((1,), (0,)), ((), ())),
                                    precision=hi, preferred_element_type=f32)
    p = jax.nn.sigmoid(praw_ref[...])
    m = acc * eemb_ref[...] * p
    out_ref[...] = jax.nn.sigmoid(jnp.sum(m, axis=1, keepdims=True))


_tc_dense = pl.pallas_call(
    _tc_body,
    out_shape=jax.ShapeDtypeStruct((BATCH, 1), jnp.float32),
)


def kernel(neighbors, start_node, end_node, path, embeds_start, embeds_end,
           embeds_path, W1, b1, W2, b2):
    nbrT = jnp.transpose(neighbors, (1, 2, 0))
    agg, eemb, praw = _sc_gather(
        nbrT, start_node.astype(jnp.int32), end_node.astype(jnp.int32),
        path.astype(jnp.int32), embeds_start, embeds_end, embeds_path)
    out = _tc_dense(agg, eemb, praw, W1, b1.reshape(1, EMBED_DIM), W2,
                    b2.reshape(1, EMBED_DIM))
    return out.reshape(BATCH)
